# Initial kernel scaffold; baseline (speedup 1.0000x reference)
import jax
import jax.numpy as jnp
from jax.experimental import pallas as pl


def kernel(imgs, points_x, points_y):
    raise NotImplementedError("write your pallas kernel here")


# speedup vs baseline: 136.2146x; 136.2146x over previous
import jax
import jax.numpy as jnp
from jax.experimental import pallas as pl
from jax.experimental.pallas import tpu as pltpu

_PATCH = 16
_NPAD = 256


def _occl_body(px_ref, py_ref, img_ref, out_ref):
    _, c, h, w = img_ref.shape
    px = px_ref[0]
    py = py_ref[0]
    hh = jax.lax.broadcasted_iota(jnp.int32, (h, _NPAD), 0)
    rowhit = jnp.logical_and(hh >= px, hh < px + _PATCH).astype(jnp.float32)
    ww = jax.lax.broadcasted_iota(jnp.int32, (_NPAD, w), 1)
    colhit = jnp.logical_and(ww >= py, ww < py + _PATCH).astype(jnp.float32)
    cov = jax.lax.dot_general(
        rowhit, colhit, (((1,), (0,)), ((), ())),
        preferred_element_type=jnp.float32)
    mask = (cov == 0.0).astype(jnp.float32)
    out_ref[...] = img_ref[...] * mask[None, None]


def _occlude(px3, py3, imgs, interpret=False):
    b, c, h, w = imgs.shape
    return pl.pallas_call(
        _occl_body,
        grid=(b,),
        in_specs=[
            pl.BlockSpec((1, 1, _NPAD), lambda i: (i, 0, 0)),
            pl.BlockSpec((1, _NPAD, 1), lambda i: (i, 0, 0)),
            pl.BlockSpec((1, c, h, w), lambda i: (i, 0, 0, 0)),
        ],
        out_specs=pl.BlockSpec((1, c, h, w), lambda i: (i, 0, 0, 0)),
        out_shape=jax.ShapeDtypeStruct(imgs.shape, imgs.dtype),
        compiler_params=pltpu.CompilerParams(
            dimension_semantics=("arbitrary",),
        ),
        interpret=interpret,
    )(px3, py3, imgs)


@jax.jit
def kernel(imgs, points_x, points_y):
    b, _, _, _ = imgs.shape
    n = points_x.shape[1]
    pad = ((0, 0), (0, _NPAD - n))
    px = jnp.pad(points_x, pad, constant_values=-100000)
    py = jnp.pad(points_y, pad, constant_values=-100000)
    px3 = px.reshape(b, 1, _NPAD)
    py3 = py.reshape(b, _NPAD, 1)
    return _occlude(px3, py3, imgs)
